# Initial kernel scaffold; baseline (speedup 1.0000x reference)
import functools

import jax
import jax.numpy as jnp
from jax import lax
from jax.experimental import pallas as pl
from jax.experimental.pallas import tpu as pltpu
from jax.experimental.pallas import tpu_sc as plsc

N = 10000
K = 32
F = 128
G = 16
EDGES = N * K

NC = 2
NS = 16
NW = NC * NS
E_PW = EDGES // NW
CH = 80
NCH = E_PW // CH

BN = 400
GRID = N // BN


def _phase1_body(feat_ref, rbf_ref, wi_ref, wf1_ref, bf1_ref, wf2_ref,
                 bf2_ref, h_ref, filt_ref):
    h_ref[...] = jnp.dot(feat_ref[...], wi_ref[...],
                         preferred_element_type=jnp.float32)
    t = jnp.tanh(jnp.dot(rbf_ref[...], wf1_ref[...],
                         preferred_element_type=jnp.float32) + bf1_ref[...])
    filt_ref[...] = jnp.dot(t, wf2_ref[...],
                            preferred_element_type=jnp.float32) + bf2_ref[...]


def _phase1(feat, rbf, w_init, wf1, bf1, wf2, bf2):
    full = lambda r, c: pl.BlockSpec((r, c), lambda i: (0, 0))
    return pl.pallas_call(
        _phase1_body,
        grid=(GRID,),
        in_specs=[
            pl.BlockSpec((BN, F), lambda i: (i, 0)),
            pl.BlockSpec((BN * K, G), lambda i: (i, 0)),
            full(F, F),
            full(G, F),
            full(1, F),
            full(F, F),
            full(1, F),
        ],
        out_specs=[
            pl.BlockSpec((BN, F), lambda i: (i, 0)),
            pl.BlockSpec((BN * K, F), lambda i: (i, 0)),
        ],
        out_shape=[
            jax.ShapeDtypeStruct((N, F), jnp.float32),
            jax.ShapeDtypeStruct((EDGES, F), jnp.float32),
        ],
    )(feat, rbf, w_init, wf1, bf1, wf2, bf2)


def _sc_gather_body(h_hbm, nl_hbm, nf_hbm, idx_v, rows_v, sem):
    wid = lax.axis_index("s") * NC + lax.axis_index("c")
    base = wid * E_PW

    def step(i, carry):
        off = base + i * CH
        pltpu.sync_copy(nl_hbm.at[pl.ds(off, CH)], idx_v)
        pltpu.async_copy(h_hbm.at[idx_v], rows_v, sem).wait()
        pltpu.sync_copy(rows_v, nf_hbm.at[pl.ds(off, CH)])
        return carry

    lax.fori_loop(0, NCH, step, 0)


_sc_gather = pl.kernel(
    _sc_gather_body,
    out_type=jax.ShapeDtypeStruct((EDGES, F), jnp.float32),
    mesh=plsc.VectorSubcoreMesh(core_axis_name="c", subcore_axis_name="s",
                                num_cores=NC, num_subcores=NS),
    scratch_types=[
        pltpu.VMEM((CH,), jnp.int32),
        pltpu.VMEM((CH, F), jnp.float32),
        pltpu.SemaphoreType.DMA,
    ],
)


def _phase3_body(nf_ref, filt_ref, wo1_ref, bo1_ref, wo2_ref, bo2_ref,
                 out_ref):
    prod = nf_ref[...] * filt_ref[...]
    conv = jnp.sum(prod.reshape(BN, K, F), axis=1)
    t = jnp.tanh(jnp.dot(conv, wo1_ref[...],
                         preferred_element_type=jnp.float32) + bo1_ref[...])
    out_ref[...] = jnp.dot(t, wo2_ref[...],
                           preferred_element_type=jnp.float32) + bo2_ref[...]


def _phase3(nf, filt, wo1, bo1, wo2, bo2):
    full = lambda r, c: pl.BlockSpec((r, c), lambda i: (0, 0))
    return pl.pallas_call(
        _phase3_body,
        grid=(GRID,),
        in_specs=[
            pl.BlockSpec((BN * K, F), lambda i: (i, 0)),
            pl.BlockSpec((BN * K, F), lambda i: (i, 0)),
            full(F, F),
            full(1, F),
            full(F, F),
            full(1, F),
        ],
        out_specs=pl.BlockSpec((BN, F), lambda i: (i, 0)),
        out_shape=jax.ShapeDtypeStruct((N, F), jnp.float32),
    )(nf, filt, wo1, bo1, wo2, bo2)


def kernel(features, rbf_expansion, neighbor_list, W_init, Wf1, bf1, Wf2,
           bf2, Wo1, bo1, Wo2, bo2):
    feat = features.reshape(N, F).astype(jnp.float32)
    rbf = rbf_expansion.reshape(EDGES, G).astype(jnp.float32)
    nl = neighbor_list.reshape(EDGES).astype(jnp.int32)

    h, filt = _phase1(feat, rbf, W_init, Wf1, bf1.reshape(1, F), Wf2,
                      bf2.reshape(1, F))
    nf = _sc_gather(h, nl)
    out = _phase3(nf, filt, Wo1, bo1.reshape(1, F), Wo2, bo2.reshape(1, F))
    return out.reshape(1, N, F), filt.reshape(1, N, K, F)


# speedup vs baseline: 3817.5688x; 3817.5688x over previous
import functools

import jax
import jax.numpy as jnp
from jax import lax
from jax.experimental import pallas as pl
from jax.experimental.pallas import tpu as pltpu
from jax.experimental.pallas import tpu_sc as plsc

N = 10000
K = 32
F = 128
G = 16
EDGES = N * K

NC = 2
NS = 16
NW = NC * NS
E_PW = EDGES // NW
CH = 80
NCH = E_PW // CH

BN = 400
GRID = N // BN


def _phase1_body(feat_ref, rbf_ref, wi_ref, wf1_ref, bf1_ref, wf2_ref,
                 bf2_ref, h_ref, filt_ref):
    h_ref[...] = jnp.dot(feat_ref[...], wi_ref[...],
                         preferred_element_type=jnp.float32)
    t = jnp.tanh(jnp.dot(rbf_ref[...], wf1_ref[...],
                         preferred_element_type=jnp.float32) + bf1_ref[...])
    filt_ref[...] = jnp.dot(t, wf2_ref[...],
                            preferred_element_type=jnp.float32) + bf2_ref[...]


def _phase1(feat, rbf, w_init, wf1, bf1, wf2, bf2):
    full = lambda r, c: pl.BlockSpec((r, c), lambda i: (0, 0))
    return pl.pallas_call(
        _phase1_body,
        grid=(GRID,),
        in_specs=[
            pl.BlockSpec((BN, F), lambda i: (i, 0)),
            pl.BlockSpec((BN * K, G), lambda i: (i, 0)),
            full(F, F),
            full(G, F),
            full(1, F),
            full(F, F),
            full(1, F),
        ],
        out_specs=[
            pl.BlockSpec((BN, F), lambda i: (i, 0)),
            pl.BlockSpec((BN * K, F), lambda i: (i, 0)),
        ],
        out_shape=[
            jax.ShapeDtypeStruct((N, F), jnp.float32),
            jax.ShapeDtypeStruct((EDGES, F), jnp.float32),
        ],
    )(feat, rbf, w_init, wf1, bf1, wf2, bf2)


def _sc_gather_body(h_hbm, nl_hbm, nf_hbm, idx_v, rows_v, sem):
    wid = lax.axis_index("s") * NC + lax.axis_index("c")
    base = wid * E_PW

    def step(i, carry):
        off = base + i * CH
        pltpu.sync_copy(nl_hbm.at[pl.ds(off, CH)], idx_v)
        pltpu.async_copy(h_hbm.at[idx_v], rows_v, sem).wait()
        pltpu.sync_copy(rows_v, nf_hbm.at[pl.ds(off, CH)])
        return carry

    lax.fori_loop(0, NCH, step, 0)


@functools.cache
def _make_sc_gather():
    return pl.kernel(
        _sc_gather_body,
        out_type=jax.ShapeDtypeStruct((EDGES, F), jnp.float32),
        mesh=plsc.VectorSubcoreMesh(core_axis_name="c", subcore_axis_name="s",
                                    num_cores=NC, num_subcores=NS),
        scratch_types=[
            pltpu.VMEM((CH,), jnp.int32),
            pltpu.VMEM((CH, F), jnp.float32),
            pltpu.SemaphoreType.DMA,
        ],
    )


def _sc_gather(h, nl):
    return _make_sc_gather()(h, nl)


def _phase3_body(nf_ref, filt_ref, wo1_ref, bo1_ref, wo2_ref, bo2_ref,
                 out_ref):
    prod = nf_ref[...] * filt_ref[...]
    conv = jnp.sum(prod.reshape(BN, K, F), axis=1)
    t = jnp.tanh(jnp.dot(conv, wo1_ref[...],
                         preferred_element_type=jnp.float32) + bo1_ref[...])
    out_ref[...] = jnp.dot(t, wo2_ref[...],
                           preferred_element_type=jnp.float32) + bo2_ref[...]


def _phase3(nf, filt, wo1, bo1, wo2, bo2):
    full = lambda r, c: pl.BlockSpec((r, c), lambda i: (0, 0))
    return pl.pallas_call(
        _phase3_body,
        grid=(GRID,),
        in_specs=[
            pl.BlockSpec((BN * K, F), lambda i: (i, 0)),
            pl.BlockSpec((BN * K, F), lambda i: (i, 0)),
            full(F, F),
            full(1, F),
            full(F, F),
            full(1, F),
        ],
        out_specs=pl.BlockSpec((BN, F), lambda i: (i, 0)),
        out_shape=jax.ShapeDtypeStruct((N, F), jnp.float32),
    )(nf, filt, wo1, bo1, wo2, bo2)


def kernel(features, rbf_expansion, neighbor_list, W_init, Wf1, bf1, Wf2,
           bf2, Wo1, bo1, Wo2, bo2):
    feat = features.reshape(N, F).astype(jnp.float32)
    rbf = rbf_expansion.reshape(EDGES, G).astype(jnp.float32)
    nl = neighbor_list.reshape(EDGES).astype(jnp.int32)

    h, filt = _phase1(feat, rbf, W_init, Wf1, bf1.reshape(1, F), Wf2,
                      bf2.reshape(1, F))
    nf = _sc_gather(h, nl)
    out = _phase3(nf, filt, Wo1, bo1.reshape(1, F), Wo2, bo2.reshape(1, F))
    return out.reshape(1, N, F), filt.reshape(1, N, K, F)


# speedup vs baseline: 4768.5553x; 1.2491x over previous
import functools

import jax
import jax.numpy as jnp
from jax import lax
from jax.experimental import pallas as pl
from jax.experimental.pallas import tpu as pltpu
from jax.experimental.pallas import tpu_sc as plsc

N = 10000
K = 32
F = 128
G = 16
EDGES = N * K

NC = 2
NS = 16
NW = NC * NS
E_PW = EDGES // NW
CH = 80
NCH = E_PW // CH

BN = 400
GRID = N // BN


def _phase1_body(feat_ref, rbf_ref, wi_ref, wf1_ref, bf1_ref, wf2_ref,
                 bf2_ref, h_ref, filt_ref):
    h_ref[...] = jnp.dot(feat_ref[...], wi_ref[...],
                         preferred_element_type=jnp.float32)
    t = jnp.tanh(jnp.dot(rbf_ref[...], wf1_ref[...],
                         preferred_element_type=jnp.float32) + bf1_ref[...])
    filt_ref[...] = jnp.dot(t, wf2_ref[...],
                            preferred_element_type=jnp.float32) + bf2_ref[...]


def _phase1(feat, rbf, w_init, wf1, bf1, wf2, bf2):
    full = lambda r, c: pl.BlockSpec((r, c), lambda i: (0, 0))
    return pl.pallas_call(
        _phase1_body,
        grid=(GRID,),
        in_specs=[
            pl.BlockSpec((BN, F), lambda i: (i, 0)),
            pl.BlockSpec((BN * K, G), lambda i: (i, 0)),
            full(F, F),
            full(G, F),
            full(1, F),
            full(F, F),
            full(1, F),
        ],
        out_specs=[
            pl.BlockSpec((BN, F), lambda i: (i, 0)),
            pl.BlockSpec((BN * K, F), lambda i: (i, 0)),
        ],
        out_shape=[
            jax.ShapeDtypeStruct((N, F), jnp.float32),
            jax.ShapeDtypeStruct((EDGES, F), jnp.float32),
        ],
    )(feat, rbf, w_init, wf1, bf1, wf2, bf2)


CN = 8
CE = CN * K
CHUNKS = N // CN
CPW_HI = 40
CPW_LO = 39
NV = F // 16


def _sc_conv_body(h_hbm, nl_hbm, filt_hbm, conv_hbm, idx_v, rows_v, filt_v,
                  acc_v, sem_g, sem_f):
    wid = lax.axis_index("s") * NC + lax.axis_index("c")
    hi = wid < 2
    cbase = jnp.where(hi, wid * CPW_HI, 2 * CPW_HI + (wid - 2) * CPW_LO)
    nch = jnp.where(hi, CPW_HI, CPW_LO)

    def chunk(i, carry):
        nstart = pl.multiple_of((cbase + i) * CN, CN)
        estart = pl.multiple_of(nstart * K, CE)
        pltpu.sync_copy(nl_hbm.at[pl.ds(estart, CE)], idx_v)
        g0 = pltpu.async_copy(h_hbm.at[idx_v.at[pl.ds(0, CE // 2)]],
                              rows_v.at[pl.ds(0, CE // 2)], sem_g)
        g1 = pltpu.async_copy(h_hbm.at[idx_v.at[pl.ds(CE // 2, CE // 2)]],
                              rows_v.at[pl.ds(CE // 2, CE // 2)], sem_g)
        fcp = pltpu.async_copy(filt_hbm.at[pl.ds(estart, CE)], filt_v, sem_f)
        g0.wait()
        g1.wait()
        fcp.wait()
        for ni in range(CN):
            def kstep(k2, accs, ni=ni):
                e = ni * K + k2
                return tuple(
                    accs[j]
                    + rows_v[e, pl.ds(16 * j, 16)]
                    * filt_v[e, pl.ds(16 * j, 16)]
                    for j in range(NV))
            accs = lax.fori_loop(
                0, K, kstep,
                tuple(jnp.zeros((16,), jnp.float32) for _ in range(NV)))
            for j in range(NV):
                acc_v[ni, pl.ds(16 * j, 16)] = accs[j]
        pltpu.sync_copy(acc_v, conv_hbm.at[pl.ds(nstart, CN)])
        return carry

    lax.fori_loop(0, nch, chunk, 0)


@functools.cache
def _make_sc_conv():
    return pl.kernel(
        _sc_conv_body,
        out_type=jax.ShapeDtypeStruct((N, F), jnp.float32),
        mesh=plsc.VectorSubcoreMesh(core_axis_name="c", subcore_axis_name="s",
                                    num_cores=NC, num_subcores=NS),
        scratch_types=[
            pltpu.VMEM((CE,), jnp.int32),
            pltpu.VMEM((CE, F), jnp.float32),
            pltpu.VMEM((CE, F), jnp.float32),
            pltpu.VMEM((CN, F), jnp.float32),
            pltpu.SemaphoreType.DMA,
            pltpu.SemaphoreType.DMA,
        ],
    )


def _sc_conv(h, nl, filt):
    return _make_sc_conv()(h, nl, filt)


def _phase3_body(conv_ref, wo1_ref, bo1_ref, wo2_ref, bo2_ref, out_ref):
    t = jnp.tanh(jnp.dot(conv_ref[...], wo1_ref[...],
                         preferred_element_type=jnp.float32) + bo1_ref[...])
    out_ref[...] = jnp.dot(t, wo2_ref[...],
                           preferred_element_type=jnp.float32) + bo2_ref[...]


def _phase3(conv, wo1, bo1, wo2, bo2):
    full = lambda r, c: pl.BlockSpec((r, c), lambda i: (0, 0))
    return pl.pallas_call(
        _phase3_body,
        grid=(5,),
        in_specs=[
            pl.BlockSpec((N // 5, F), lambda i: (i, 0)),
            full(F, F),
            full(1, F),
            full(F, F),
            full(1, F),
        ],
        out_specs=pl.BlockSpec((N // 5, F), lambda i: (i, 0)),
        out_shape=jax.ShapeDtypeStruct((N, F), jnp.float32),
    )(conv, wo1, bo1, wo2, bo2)


def kernel(features, rbf_expansion, neighbor_list, W_init, Wf1, bf1, Wf2,
           bf2, Wo1, bo1, Wo2, bo2):
    feat = features.reshape(N, F).astype(jnp.float32)
    rbf = rbf_expansion.reshape(EDGES, G).astype(jnp.float32)
    nl = neighbor_list.reshape(EDGES).astype(jnp.int32)

    h, filt = _phase1(feat, rbf, W_init, Wf1, bf1.reshape(1, F), Wf2,
                      bf2.reshape(1, F))
    conv = _sc_conv(h, nl, filt)
    out = _phase3(conv, Wo1, bo1.reshape(1, F), Wo2, bo2.reshape(1, F))
    return out.reshape(1, N, F), filt.reshape(1, N, K, F)


# speedup vs baseline: 6094.1811x; 1.2780x over previous
import functools

import jax
import jax.numpy as jnp
from jax import lax
from jax.experimental import pallas as pl
from jax.experimental.pallas import tpu as pltpu
from jax.experimental.pallas import tpu_sc as plsc

N = 10000
K = 32
F = 128
G = 16
EDGES = N * K

NC = 2
NS = 16
NW = NC * NS
E_PW = EDGES // NW
CH = 80
NCH = E_PW // CH

BN = 400
GRID = N // BN


def _phase1_body(feat_ref, rbf_ref, wi_ref, wf1_ref, bf1_ref, wf2_ref,
                 bf2_ref, h_ref, filt_ref):
    h_ref[...] = jnp.dot(feat_ref[...], wi_ref[...],
                         preferred_element_type=jnp.float32)
    t = jnp.tanh(jnp.dot(rbf_ref[...], wf1_ref[...],
                         preferred_element_type=jnp.float32) + bf1_ref[...])
    filt_ref[...] = jnp.dot(t, wf2_ref[...],
                            preferred_element_type=jnp.float32) + bf2_ref[...]


def _phase1(feat, rbf, w_init, wf1, bf1, wf2, bf2):
    full = lambda r, c: pl.BlockSpec((r, c), lambda i: (0, 0))
    return pl.pallas_call(
        _phase1_body,
        grid=(GRID,),
        in_specs=[
            pl.BlockSpec((BN, F), lambda i: (i, 0)),
            pl.BlockSpec((BN * K, G), lambda i: (i, 0)),
            full(F, F),
            full(G, F),
            full(1, F),
            full(F, F),
            full(1, F),
        ],
        out_specs=[
            pl.BlockSpec((BN, F), lambda i: (i, 0)),
            pl.BlockSpec((BN * K, F), lambda i: (i, 0)),
        ],
        out_shape=[
            jax.ShapeDtypeStruct((N, F), jnp.float32),
            jax.ShapeDtypeStruct((EDGES, F), jnp.float32),
        ],
    )(feat, rbf, w_init, wf1, bf1, wf2, bf2)


GN = 8
GE = GN * K
SE = GE // 2
GROUPS = N // GN
GPW_HI = 40
GPW_LO = 39
IDX_LO = GPW_LO * GE
NV = F // 16


def _sc_conv_body(h_hbm, nl_hbm, filt_hbm, conv_hbm, idx_all, rows_v, filt_v,
                  acc_v, sem_g0, sem_g1, sem_f0, sem_f1):
    wid = lax.axis_index("s") * NC + lax.axis_index("c")
    hi = wid < 2
    gbase = jnp.where(hi, wid * GPW_HI, 2 * GPW_HI + (wid - 2) * GPW_LO)
    ng = jnp.where(hi, GPW_HI, GPW_LO)
    ns = 2 * ng
    ebase = pl.multiple_of(gbase * GE, GE)

    pltpu.sync_copy(nl_hbm.at[pl.ds(ebase, IDX_LO)],
                    idx_all.at[pl.ds(0, IDX_LO)])

    @pl.when(hi)
    def _():
        pltpu.sync_copy(nl_hbm.at[pl.ds(ebase + IDX_LO, GE)],
                        idx_all.at[pl.ds(IDX_LO, GE)])

    sem_g = (sem_g0, sem_g1)
    sem_f = (sem_f0, sem_f1)

    def issue(s, b):
        le = pl.multiple_of(s * SE, SE)
        pltpu.async_copy(h_hbm.at[idx_all.at[pl.ds(le, SE)]],
                         rows_v.at[b], sem_g[b])
        pltpu.async_copy(filt_hbm.at[pl.ds(ebase + le, SE)],
                         filt_v.at[b], sem_f[b])

    def wait(b):
        pltpu.make_async_copy(h_hbm.at[pl.ds(0, SE)], rows_v.at[b],
                              sem_g[b]).wait()
        pltpu.make_async_copy(filt_hbm.at[pl.ds(0, SE)], filt_v.at[b],
                              sem_f[b]).wait()

    issue(0, 0)

    def group(i, carry):
        for b in (0, 1):
            s = 2 * i + b

            @pl.when(s < ns)
            def _(b=b, s=s):
                wait(b)

                @pl.when(s + 1 < ns)
                def _():
                    issue(s + 1, 1 - b)

                for ni in range(GN // 2):
                    def kstep(k2, accs, ni=ni):
                        e = ni * K + k2
                        return tuple(
                            accs[j]
                            + rows_v[b, e, pl.ds(16 * j, 16)]
                            * filt_v[b, e, pl.ds(16 * j, 16)]
                            for j in range(NV))
                    accs = lax.fori_loop(
                        0, K, kstep,
                        tuple(jnp.zeros((16,), jnp.float32)
                              for _ in range(NV)))
                    for j in range(NV):
                        acc_v[b * (GN // 2) + ni, pl.ds(16 * j, 16)] = accs[j]

        @pl.when(i < ng)
        def _():
            nstart = pl.multiple_of((gbase + i) * GN, GN)
            pltpu.sync_copy(acc_v, conv_hbm.at[pl.ds(nstart, GN)])

        return carry

    lax.fori_loop(0, GPW_HI, group, 0)


@functools.cache
def _make_sc_conv():
    return pl.kernel(
        _sc_conv_body,
        out_type=jax.ShapeDtypeStruct((N, F), jnp.float32),
        mesh=plsc.VectorSubcoreMesh(core_axis_name="c", subcore_axis_name="s",
                                    num_cores=NC, num_subcores=NS),
        scratch_types=[
            pltpu.VMEM((GPW_HI * GE,), jnp.int32),
            pltpu.VMEM((2, SE, F), jnp.float32),
            pltpu.VMEM((2, SE, F), jnp.float32),
            pltpu.VMEM((GN, F), jnp.float32),
            pltpu.SemaphoreType.DMA,
            pltpu.SemaphoreType.DMA,
            pltpu.SemaphoreType.DMA,
            pltpu.SemaphoreType.DMA,
        ],
    )


def _sc_conv(h, nl, filt):
    return _make_sc_conv()(h, nl, filt)


def _phase3_body(conv_ref, wo1_ref, bo1_ref, wo2_ref, bo2_ref, out_ref):
    t = jnp.tanh(jnp.dot(conv_ref[...], wo1_ref[...],
                         preferred_element_type=jnp.float32) + bo1_ref[...])
    out_ref[...] = jnp.dot(t, wo2_ref[...],
                           preferred_element_type=jnp.float32) + bo2_ref[...]


def _phase3(conv, wo1, bo1, wo2, bo2):
    full = lambda r, c: pl.BlockSpec((r, c), lambda i: (0, 0))
    return pl.pallas_call(
        _phase3_body,
        grid=(5,),
        in_specs=[
            pl.BlockSpec((N // 5, F), lambda i: (i, 0)),
            full(F, F),
            full(1, F),
            full(F, F),
            full(1, F),
        ],
        out_specs=pl.BlockSpec((N // 5, F), lambda i: (i, 0)),
        out_shape=jax.ShapeDtypeStruct((N, F), jnp.float32),
    )(conv, wo1, bo1, wo2, bo2)


def kernel(features, rbf_expansion, neighbor_list, W_init, Wf1, bf1, Wf2,
           bf2, Wo1, bo1, Wo2, bo2):
    feat = features.reshape(N, F).astype(jnp.float32)
    rbf = rbf_expansion.reshape(EDGES, G).astype(jnp.float32)
    nl = neighbor_list.reshape(EDGES).astype(jnp.int32)

    h, filt = _phase1(feat, rbf, W_init, Wf1, bf1.reshape(1, F), Wf2,
                      bf2.reshape(1, F))
    conv = _sc_conv(h, nl, filt)
    out = _phase3(conv, Wo1, bo1.reshape(1, F), Wo2, bo2.reshape(1, F))
    return out.reshape(1, N, F), filt.reshape(1, N, K, F)


# speedup vs baseline: 6965.4136x; 1.1430x over previous
import functools

import jax
import jax.numpy as jnp
from jax import lax
from jax.experimental import pallas as pl
from jax.experimental.pallas import tpu as pltpu
from jax.experimental.pallas import tpu_sc as plsc

N = 10000
K = 32
F = 128
G = 16
EDGES = N * K

NC = 2
NS = 16
NW = NC * NS
E_PW = EDGES // NW
CH = 80
NCH = E_PW // CH

BN = 400
GRID = N // BN


def _phase1_body(feat_ref, rbf_ref, wi_ref, wf1_ref, bf1_ref, wf2_ref,
                 bf2_ref, filt_ref, h_ref):
    h_ref[...] = jnp.dot(feat_ref[...], wi_ref[...],
                         preferred_element_type=jnp.float32)
    t = jnp.tanh(jnp.dot(rbf_ref[...], wf1_ref[...],
                         preferred_element_type=jnp.float32) + bf1_ref[...])
    filt_ref[...] = jnp.dot(t, wf2_ref[...],
                            preferred_element_type=jnp.float32) + bf2_ref[...]


def _phase1(feat, rbf, w_init, wf1, bf1, wf2, bf2):
    full = lambda r, c: pl.BlockSpec((r, c), lambda i: (0, 0))
    return pl.pallas_call(
        _phase1_body,
        grid=(GRID,),
        in_specs=[
            pl.BlockSpec((BN, F), lambda i: (i, 0)),
            pl.BlockSpec((BN * K, G), lambda i: (i, 0)),
            full(F, F),
            full(G, F),
            full(1, F),
            full(F, F),
            full(1, F),
        ],
        out_specs=[
            pl.BlockSpec((BN * K, F), lambda i: (i, 0)),
            pl.BlockSpec((BN, F), lambda i: (i, 0)),
        ],
        out_shape=[
            jax.ShapeDtypeStruct((EDGES, F), jnp.float32),
            jax.ShapeDtypeStruct((N, F), jnp.float32),
        ],
    )(feat, rbf, w_init, wf1, bf1, wf2, bf2)


GN = 8
GE = GN * K
SE = GE // 2
GROUPS = N // GN
GPW_HI = 40
GPW_LO = 39
IDX_LO = GPW_LO * GE
NV = F // 16
NSLOT = 3
TRIOS = (2 * GPW_HI + NSLOT - 1) // NSLOT


def _sc_conv_body(h_hbm, nl_hbm, filt_hbm, conv_hbm, idx_all, rows_v, filt_v,
                  acc_v, sem_g0, sem_g1, sem_g2, sem_f0, sem_f1, sem_f2,
                  sem_w):
    wid = lax.axis_index("s") * NC + lax.axis_index("c")
    hi = wid < 2
    gbase = jnp.where(hi, wid * GPW_HI, 2 * GPW_HI + (wid - 2) * GPW_LO)
    ng = jnp.where(hi, GPW_HI, GPW_LO)
    ns = 2 * ng
    ebase = pl.multiple_of(gbase * GE, GE)

    pltpu.sync_copy(nl_hbm.at[pl.ds(ebase, IDX_LO)],
                    idx_all.at[pl.ds(0, IDX_LO)])

    @pl.when(hi)
    def _():
        pltpu.sync_copy(nl_hbm.at[pl.ds(ebase + IDX_LO, GE)],
                        idx_all.at[pl.ds(IDX_LO, GE)])

    sem_g = (sem_g0, sem_g1, sem_g2)
    sem_f = (sem_f0, sem_f1, sem_f2)

    def issue(s, b):
        le = pl.multiple_of(s * SE, SE)
        pltpu.async_copy(h_hbm.at[idx_all.at[pl.ds(le, SE)]],
                         rows_v.at[b], sem_g[b])
        pltpu.async_copy(filt_hbm.at[pl.ds(ebase + le, SE)],
                         filt_v.at[b], sem_f[b])

    def wait(b):
        pltpu.make_async_copy(h_hbm.at[pl.ds(0, SE)], rows_v.at[b],
                              sem_g[b]).wait()
        pltpu.make_async_copy(filt_hbm.at[pl.ds(0, SE)], filt_v.at[b],
                              sem_f[b]).wait()

    def wait_conv_write():
        pltpu.make_async_copy(acc_v, conv_hbm.at[pl.ds(0, GN)],
                              sem_w).wait()

    issue(0, 0)
    issue(1, 1)

    def trio(i, carry):
        for b in range(NSLOT):
            s = 3 * i + b

            @pl.when(s < ns)
            def _(b=b, s=s):
                wait(b)

                @pl.when(s + 2 < ns)
                def _():
                    issue(s + 2, (b + 2) % NSLOT)

                half = s % 2
                @pl.when((half == 0) & (s > 0))
                def _():
                    wait_conv_write()

                for ni in range(GN // 2):
                    def kstep(k2, accs, ni=ni):
                        e = ni * K + k2
                        return tuple(
                            accs[j]
                            + rows_v[b, e, pl.ds(16 * j, 16)]
                            * filt_v[b, e, pl.ds(16 * j, 16)]
                            for j in range(NV))
                    accs = lax.fori_loop(
                        0, K, kstep,
                        tuple(jnp.zeros((16,), jnp.float32)
                              for _ in range(NV)))
                    row = half * (GN // 2) + ni
                    for j in range(NV):
                        acc_v[row, pl.ds(16 * j, 16)] = accs[j]

                @pl.when(half == 1)
                def _():
                    g = lax.div(s, 2)
                    nstart = pl.multiple_of((gbase + g) * GN, GN)
                    pltpu.async_copy(acc_v, conv_hbm.at[pl.ds(nstart, GN)],
                                     sem_w)

        return carry

    lax.fori_loop(0, TRIOS, trio, 0)
    wait_conv_write()


@functools.cache
def _make_sc_conv():
    return pl.kernel(
        _sc_conv_body,
        out_type=jax.ShapeDtypeStruct((N, F), jnp.float32),
        mesh=plsc.VectorSubcoreMesh(core_axis_name="c", subcore_axis_name="s",
                                    num_cores=NC, num_subcores=NS),
        scratch_types=[
            pltpu.VMEM((GPW_HI * GE,), jnp.int32),
            pltpu.VMEM((NSLOT, SE, F), jnp.float32),
            pltpu.VMEM((NSLOT, SE, F), jnp.float32),
            pltpu.VMEM((GN, F), jnp.float32),
            pltpu.SemaphoreType.DMA,
            pltpu.SemaphoreType.DMA,
            pltpu.SemaphoreType.DMA,
            pltpu.SemaphoreType.DMA,
            pltpu.SemaphoreType.DMA,
            pltpu.SemaphoreType.DMA,
            pltpu.SemaphoreType.DMA,
        ],
    )


def _sc_conv(h, nl, filt):
    return _make_sc_conv()(h, nl, filt)


def _phase3_body(conv_ref, wo1_ref, bo1_ref, wo2_ref, bo2_ref, out_ref):
    t = jnp.tanh(jnp.dot(conv_ref[...], wo1_ref[...],
                         preferred_element_type=jnp.float32) + bo1_ref[...])
    out_ref[...] = jnp.dot(t, wo2_ref[...],
                           preferred_element_type=jnp.float32) + bo2_ref[...]


def _phase3(conv, wo1, bo1, wo2, bo2):
    full = lambda r, c: pl.BlockSpec((r, c), lambda i: (0, 0))
    return pl.pallas_call(
        _phase3_body,
        grid=(5,),
        in_specs=[
            pl.BlockSpec((N // 5, F), lambda i: (i, 0)),
            full(F, F),
            full(1, F),
            full(F, F),
            full(1, F),
        ],
        out_specs=pl.BlockSpec((N // 5, F), lambda i: (i, 0)),
        out_shape=jax.ShapeDtypeStruct((N, F), jnp.float32),
    )(conv, wo1, bo1, wo2, bo2)


def kernel(features, rbf_expansion, neighbor_list, W_init, Wf1, bf1, Wf2,
           bf2, Wo1, bo1, Wo2, bo2):
    feat = features.reshape(N, F).astype(jnp.float32)
    rbf = rbf_expansion.reshape(EDGES, G).astype(jnp.float32)
    nl = neighbor_list.reshape(EDGES).astype(jnp.int32)

    filt, h = _phase1(feat, rbf, W_init, Wf1, bf1.reshape(1, F), Wf2,
                      bf2.reshape(1, F))
    conv = _sc_conv(h, nl, filt)
    out = _phase3(conv, Wo1, bo1.reshape(1, F), Wo2, bo2.reshape(1, F))
    return out.reshape(1, N, F), filt.reshape(1, N, K, F)
